# Initial kernel scaffold; baseline (speedup 1.0000x reference)
import jax
import jax.numpy as jnp
from jax.experimental import pallas as pl


def kernel(x, edge_index, batch, W1, b1, W2, b2, Wl, bl):
    raise NotImplementedError("write your pallas kernel here")


# speedup vs baseline: 14.4449x; 14.4449x over previous
import functools

import jax
import jax.numpy as jnp
from jax import lax
from jax.experimental import pallas as pl
from jax.experimental.pallas import tpu as pltpu
from jax.experimental.pallas import tpu_sc as plsc

N = 10000
E = 320000
D = 128
H = 128
C = 2
G = 64

NC = 2
NS = 16
NW = NC * NS
EPW = E // NW
K = 80
NCHUNK = EPW // K
N_ACC = 10240
RPT = N_ACC // NS
ZR = 128

N_PAD = 10240
RB = 1024
NG = N_PAD // RB

_MESH = plsc.VectorSubcoreMesh(core_axis_name="c", subcore_axis_name="s")


def _deg_body(dst_hbm, out_hbm, didx, degbuf):
    c = lax.axis_index("c")
    s = lax.axis_index("s")
    wid = c * NS + s
    zero16 = jnp.zeros((16,), jnp.float32)
    ones16 = jnp.ones((16,), jnp.float32)

    def zfill(i, _):
        degbuf[pl.ds(i * 16, 16)] = zero16
        return 0

    lax.fori_loop(0, N_PAD // 16, zfill, 0)
    pltpu.sync_copy(dst_hbm.at[pl.ds(wid * EPW, EPW)], didx)

    def count(i, _):
        idx = didx[pl.ds(i * 16, 16)]
        plsc.addupdate_scatter(degbuf, [idx], ones16)
        return 0

    lax.fori_loop(0, EPW // 16, count, 0)
    pltpu.sync_copy(degbuf, out_hbm.at[pl.ds(wid * N_PAD, N_PAD)])


_deg_call = functools.partial(
    pl.kernel,
    out_type=jax.ShapeDtypeStruct((NW * N_PAD,), jnp.float32),
    mesh=_MESH,
    scratch_types=[
        pltpu.VMEM((EPW,), jnp.int32),
        pltpu.VMEM((N_PAD,), jnp.float32),
    ],
    compiler_params=pltpu.CompilerParams(needs_layout_passes=False),
)(_deg_body)


def _edge_body(y_hbm, src_hbm, dst_hbm, out_hbm, sidx, didx, rows, zbuf, acc, sem):
    c = lax.axis_index("c")
    s = lax.axis_index("s")
    wid = c * NS + s
    zero16 = jnp.zeros((16,), jnp.float32)

    def zrow(r, _):
        for c8 in range(H // 16):
            zbuf[r, pl.ds(c8 * 16, 16)] = zero16
        return 0

    lax.fori_loop(0, ZR, zrow, 0)
    for rep in range(RPT // ZR):
        pltpu.sync_copy(zbuf, acc.at[pl.ds(s * RPT + rep * ZR, ZR)])
    plsc.subcore_barrier()

    def chunk(i, _):
        base = wid * EPW + i * K
        pltpu.sync_copy(src_hbm.at[pl.ds(base, K)], sidx)
        pltpu.sync_copy(dst_hbm.at[pl.ds(base, K)], didx)
        pltpu.async_copy(y_hbm.at[sidx], rows, sem).wait()
        pltpu.sync_copy(rows, acc.at[didx], add=True)
        return 0

    lax.fori_loop(0, NCHUNK, chunk, 0)
    plsc.subcore_barrier()
    pltpu.sync_copy(acc.at[pl.ds(s * RPT, RPT)], out_hbm.at[c, pl.ds(s * RPT, RPT)])


_edge_call = functools.partial(
    pl.kernel,
    out_type=jax.ShapeDtypeStruct((NC, N_ACC, H), jnp.float32),
    mesh=_MESH,
    scratch_types=[
        pltpu.VMEM((K,), jnp.int32),
        pltpu.VMEM((K,), jnp.int32),
        pltpu.VMEM((K, H), jnp.float32),
        pltpu.VMEM((ZR, H), jnp.float32),
        pltpu.VMEM_SHARED((N_ACC, H), jnp.float32),
        pltpu.SemaphoreType.DMA,
    ],
)(_edge_body)


def _dinv_col(dp_ref):
    i = pl.program_id(0)
    dp = dp_ref[:, pl.ds(i * RB, RB)]
    ones = jnp.ones((NW, 1), jnp.float32)
    deg = lax.dot_general(dp, ones, (((0,), (0,)), ((), ())),
                          preferred_element_type=jnp.float32) + 1.0
    return lax.rsqrt(deg)


def _scale_mm_body(x_ref, w_ref, dp_ref, y_ref):
    dinv = _dinv_col(dp_ref)
    xw = jnp.dot(x_ref[...], w_ref[...], preferred_element_type=jnp.float32)
    y_ref[...] = xw * dinv


def _mid_body(p_ref, y1_ref, dp_ref, b1_ref, w2_ref, y2_ref):
    dinv = _dinv_col(dp_ref)
    ssum = p_ref[0] + p_ref[1] + y1_ref[...]
    h1 = jnp.maximum(dinv * ssum + b1_ref[...], 0.0)
    y2_ref[...] = jnp.dot(h1, w2_ref[...], preferred_element_type=jnp.float32) * dinv


def _final_body(p_ref, y2_ref, dp_ref, b2_ref, bf_ref, wl_ref, bl_ref,
                out_ref, ps_ref, cnt_ref):
    i = pl.program_id(0)

    @pl.when(i == 0)
    def _():
        ps_ref[...] = jnp.zeros((G, H), jnp.float32)
        cnt_ref[...] = jnp.zeros((G, H), jnp.float32)

    dinv = _dinv_col(dp_ref)
    h2 = jnp.maximum(dinv * (p_ref[0] + p_ref[1] + y2_ref[...]) + b2_ref[...], 0.0)
    gids = lax.broadcasted_iota(jnp.int32, (RB, G), 1)
    ridx = lax.broadcasted_iota(jnp.int32, (RB, G), 0) + i * RB
    onehot = ((bf_ref[...] == gids) & (ridx < N)).astype(jnp.float32)
    tdot = lambda a, b: lax.dot_general(a, b, (((0,), (0,)), ((), ())),
                                        preferred_element_type=jnp.float32)
    ps_ref[...] += tdot(onehot, h2)
    cnt_ref[...] += tdot(onehot, jnp.ones((RB, H), jnp.float32))

    @pl.when(i == NG - 1)
    def _():
        pooled = ps_ref[...] / jnp.maximum(cnt_ref[...], 1.0)
        out_ref[...] = jnp.dot(pooled, wl_ref[...],
                               preferred_element_type=jnp.float32) + bl_ref[...]


_scale_mm = pl.pallas_call(
    _scale_mm_body,
    grid=(NG,),
    in_specs=[
        pl.BlockSpec((RB, D), lambda i: (i, 0)),
        pl.BlockSpec((D, H), lambda i: (0, 0)),
        pl.BlockSpec((NW, N_PAD), lambda i: (0, 0)),
    ],
    out_specs=pl.BlockSpec((RB, H), lambda i: (i, 0)),
    out_shape=jax.ShapeDtypeStruct((N, H), jnp.float32),
)

_mid = pl.pallas_call(
    _mid_body,
    grid=(NG,),
    in_specs=[
        pl.BlockSpec((NC, RB, H), lambda i: (0, i, 0)),
        pl.BlockSpec((RB, H), lambda i: (i, 0)),
        pl.BlockSpec((NW, N_PAD), lambda i: (0, 0)),
        pl.BlockSpec((1, H), lambda i: (0, 0)),
        pl.BlockSpec((H, H), lambda i: (0, 0)),
    ],
    out_specs=pl.BlockSpec((RB, H), lambda i: (i, 0)),
    out_shape=jax.ShapeDtypeStruct((N, H), jnp.float32),
)

_final = pl.pallas_call(
    _final_body,
    grid=(NG,),
    in_specs=[
        pl.BlockSpec((NC, RB, H), lambda i: (0, i, 0)),
        pl.BlockSpec((RB, H), lambda i: (i, 0)),
        pl.BlockSpec((NW, N_PAD), lambda i: (0, 0)),
        pl.BlockSpec((1, H), lambda i: (0, 0)),
        pl.BlockSpec((RB, 1), lambda i: (i, 0)),
        pl.BlockSpec((H, H), lambda i: (0, 0)),
        pl.BlockSpec((1, H), lambda i: (0, 0)),
    ],
    out_specs=pl.BlockSpec((G, H), lambda i: (0, 0)),
    out_shape=jax.ShapeDtypeStruct((G, H), jnp.float32),
    scratch_shapes=[
        pltpu.VMEM((G, H), jnp.float32),
        pltpu.VMEM((G, H), jnp.float32),
    ],
)


def kernel(x, edge_index, batch, W1, b1, W2, b2, Wl, bl):
    src = edge_index[0].astype(jnp.int32)
    dst = edge_index[1].astype(jnp.int32)
    batch_col = batch.astype(jnp.int32).reshape(N, 1)
    b1r = b1.reshape(1, H)
    b2r = b2.reshape(1, H)
    wl_pad = jnp.zeros((H, H), jnp.float32).at[:, :C].set(Wl)
    bl_pad = jnp.zeros((1, H), jnp.float32).at[:, :C].set(bl)

    deg_parts = _deg_call(dst).reshape(NW, N_PAD)
    y1 = _scale_mm(x, W1, deg_parts)
    p1 = _edge_call(y1, src, dst)
    y2 = _mid(p1, y1, deg_parts, b1r, W2)
    p2 = _edge_call(y2, src, dst)
    out = _final(p2, y2, deg_parts, b2r, batch_col, wl_pad, bl_pad)
    return out[:, :C]
